# baseline (speedup 1.0000x reference)
import jax
import jax.numpy as jnp
from jax import lax
from jax.experimental import pallas as pl
from jax.experimental.pallas import tpu as pltpu
from jax.experimental.pallas import tpu_sc as plsc

NC = 2
NS = 16
L = 16
NW = NC * NS
B = 16384
D = 64
BPW = B // NW


def _mf_body(user_hbm, item_hbm, uw_hbm, iw_hbm, out_hbm,
             uidx_v, iidx_v, urows_v, irows_v, out_v, sem_u, sem_i):
    wid = lax.axis_index("s") * NC + lax.axis_index("c")
    base = wid * BPW
    pltpu.sync_copy(user_hbm.at[pl.ds(base, BPW)], uidx_v)
    pltpu.sync_copy(item_hbm.at[pl.ds(base, BPW)], iidx_v)
    cu = pltpu.async_copy(uw_hbm.at[uidx_v], urows_v, sem_u)
    ci = pltpu.async_copy(iw_hbm.at[iidx_v], irows_v, sem_i)
    cu.wait()
    ci.wait()
    lane = lax.iota(jnp.int32, L)
    for g in range(BPW // L):
        rowidx = g * L + lane

        def t_body(t, acc):
            col = lax.bitwise_and(lane + t, D - 1)
            a = plsc.load_gather(urows_v, [rowidx, col])
            b = plsc.load_gather(irows_v, [rowidx, col])
            return acc + a * b

        acc = lax.fori_loop(0, D, t_body, jnp.zeros((L,), jnp.float32))
        out_v[pl.ds(g * L, L)] = acc
    pltpu.sync_copy(out_v, out_hbm.at[pl.ds(base, BPW)])


def kernel(user, item, user_emb_w, item_emb_w):
    mesh = plsc.VectorSubcoreMesh(core_axis_name="c", subcore_axis_name="s")
    f = pl.kernel(
        _mf_body,
        out_type=jax.ShapeDtypeStruct((B,), jnp.float32),
        mesh=mesh,
        scratch_types=[
            pltpu.VMEM((BPW,), jnp.int32),
            pltpu.VMEM((BPW,), jnp.int32),
            pltpu.VMEM((BPW, D), jnp.float32),
            pltpu.VMEM((BPW, D), jnp.float32),
            pltpu.VMEM((BPW,), jnp.float32),
            pltpu.SemaphoreType.DMA,
            pltpu.SemaphoreType.DMA,
        ],
        compiler_params=pltpu.CompilerParams(
            needs_layout_passes=False, use_tc_tiling_on_sc=False),
    )
    return f(user.astype(jnp.int32), item.astype(jnp.int32),
             user_emb_w, item_emb_w)


# speedup vs baseline: 2.1866x; 2.1866x over previous
import jax
import jax.numpy as jnp
from jax import lax
from jax.experimental import pallas as pl
from jax.experimental.pallas import tpu as pltpu
from jax.experimental.pallas import tpu_sc as plsc

NC = 2
NS = 16
L = 16
NW = NC * NS
B = 16384
D = 64
BPW = B // NW
CH = 32
NCH = BPW // CH


def _mf_body(user_hbm, item_hbm, uw_hbm, iw_hbm, out_hbm,
             uidx_v, iidx_v, du_v, di_v, out_v, sem):
    wid = lax.axis_index("s") * NC + lax.axis_index("c")
    base = wid * BPW
    pltpu.sync_copy(user_hbm.at[pl.ds(base, BPW)], uidx_v)
    pltpu.sync_copy(item_hbm.at[pl.ds(base, BPW)], iidx_v)
    lane = lax.iota(jnp.int32, L)

    def chunk_body(c, carry):
        for g in range(CH // L):
            uvec = uidx_v[pl.ds(c * CH + g * L, L)]
            ivec = iidx_v[pl.ds(c * CH + g * L, L)]
            for j in range(L):
                r_u = uvec[j]
                r_i = ivec[j]
                pltpu.async_copy(uw_hbm.at[r_u >> 3], du_v.at[g * L + j], sem)
                pltpu.async_copy(iw_hbm.at[r_i >> 3], di_v.at[g * L + j], sem)
        for j in range(2 * CH):
            pltpu.make_async_copy(uw_hbm.at[0], du_v.at[0], sem).wait()
        for g in range(CH // L):
            uvec = uidx_v[pl.ds(c * CH + g * L, L)]
            ivec = iidx_v[pl.ds(c * CH + g * L, L)]
            su = lax.bitwise_and(uvec, 7)
            si = lax.bitwise_and(ivec, 7)
            bvec = g * L + lane

            def t_body(t, acc):
                col = lax.bitwise_and(lane + t, D - 1)
                a = plsc.load_gather(du_v, [bvec, su, col])
                b = plsc.load_gather(di_v, [bvec, si, col])
                return acc + a * b

            acc = lax.fori_loop(0, D, t_body, jnp.zeros((L,), jnp.float32))
            out_v[pl.ds(c * CH + g * L, L)] = acc
        return carry

    lax.fori_loop(0, NCH, chunk_body, 0)
    pltpu.sync_copy(out_v, out_hbm.at[pl.ds(base, BPW)])


def kernel(user, item, user_emb_w, item_emb_w):
    mesh = plsc.VectorSubcoreMesh(core_axis_name="c", subcore_axis_name="s")
    f = pl.kernel(
        _mf_body,
        out_type=jax.ShapeDtypeStruct((B,), jnp.float32),
        mesh=mesh,
        scratch_types=[
            pltpu.VMEM((BPW,), jnp.int32),
            pltpu.VMEM((BPW,), jnp.int32),
            pltpu.VMEM((CH, 8, D), jnp.float32),
            pltpu.VMEM((CH, 8, D), jnp.float32),
            pltpu.VMEM((BPW,), jnp.float32),
            pltpu.SemaphoreType.DMA,
        ],
        compiler_params=pltpu.CompilerParams(needs_layout_passes=False),
    )
    nq = user_emb_w.shape[0] // 8
    return f(user.astype(jnp.int32), item.astype(jnp.int32),
             user_emb_w.reshape(nq, 8, D), item_emb_w.reshape(nq, 8, D))
